# baseline (speedup 1.0000x reference)
import functools

import jax
import jax.numpy as jnp
from jax import lax
from jax.experimental import pallas as pl
from jax.experimental.pallas import tpu as pltpu
from jax.experimental.pallas import tpu_sc as plsc

N = 10000
E = 320000
D = 128

NC = 2
NS = 16
LANES = 16
NW = NC * NS
FPW = D // NW
EPW = E // NW
ECH = 2000
NECH = E // ECH
CPW = EPW // ECH
EGRP = ECH // LANES

_sc_params = pltpu.CompilerParams(
    use_tc_tiling_on_sc=False, needs_layout_passes=False
)

_mesh = plsc.VectorSubcoreMesh(
    core_axis_name="c", subcore_axis_name="s", num_cores=NC, num_subcores=NS
)


def _prep_body(h_ref, w_ref, a_ref, zt_ref, s_ref):
    z = jnp.dot(h_ref[...], w_ref[...], preferred_element_type=jnp.float32)
    zt_ref[...] = jnp.reshape(z.T, (NW, FPW, N))
    a2 = jnp.concatenate([a_ref[0:D, :], a_ref[D:2 * D, :]], axis=1)
    s_ref[...] = jnp.dot(z, a2, preferred_element_type=jnp.float32)


def _tc_prep(h, W, a):
    return pl.pallas_call(
        _prep_body,
        out_shape=[
            jax.ShapeDtypeStruct((NW, FPW, N), jnp.float32),
            jax.ShapeDtypeStruct((N, 2), jnp.float32),
        ],
    )(h, W, a)


@functools.partial(
    pl.kernel,
    out_type=(
        jax.ShapeDtypeStruct((NECH, 2, ECH), jnp.int32),
        jax.ShapeDtypeStruct((NW, 1, N), jnp.float32),
    ),
    mesh=_mesh,
    scratch_types=(
        pltpu.VMEM((ECH,), jnp.int32),
        pltpu.VMEM((ECH,), jnp.int32),
        pltpu.VMEM((2, ECH), jnp.int32),
        pltpu.VMEM((N, 2), jnp.float32),
        pltpu.VMEM((1, N), jnp.float32),
    ),
    compiler_params=_sc_params,
)
def _sc_weights(ei_hbm, s12_hbm, pk_out, den_out,
                src_v, dst_v, pk_v, s12_v, den_v):
    c = lax.axis_index("c")
    s = lax.axis_index("s")
    wid = c * NS + s
    base = wid * EPW

    pltpu.sync_copy(s12_hbm, s12_v)

    zero16 = jnp.zeros((LANES,), jnp.float32)

    @pl.loop(0, N // LANES)
    def _zero_den(i):
        den_v[0, pl.ds(i * LANES, LANES)] = zero16

    col0 = jnp.zeros((LANES,), jnp.int32)
    col1 = jnp.ones((LANES,), jnp.int32)

    for k in range(CPW):
        pltpu.sync_copy(ei_hbm.at[0, pl.ds(base + k * ECH, ECH)], src_v)
        pltpu.sync_copy(ei_hbm.at[1, pl.ds(base + k * ECH, ECH)], dst_v)

        @pl.loop(0, EGRP)
        def _pass1(g):
            sl = pl.ds(g * LANES, LANES)
            si = src_v[sl]
            di = dst_v[sl]
            e = (plsc.load_gather(s12_v, [si, col0])
                 + plsc.load_gather(s12_v, [di, col1]))
            e = jnp.where(e >= 0.0, e, 0.01 * e)
            w = jnp.exp(e)
            pk_v[0, sl] = lax.bitwise_or(si, lax.shift_left(di, 14))
            pk_v[1, sl] = plsc.bitcast(w, jnp.int32)
            plsc.addupdate_scatter(den_v, [col0, di], w)

        pltpu.sync_copy(pk_v, pk_out.at[wid * CPW + k])

    pltpu.sync_copy(den_v, den_out.at[wid])


@functools.partial(
    pl.kernel,
    out_type=jax.ShapeDtypeStruct((NW, FPW, N), jnp.float32),
    mesh=_mesh,
    scratch_types=(
        pltpu.VMEM((FPW, N), jnp.float32),
        pltpu.VMEM((FPW // 2, N), jnp.int32),
        pltpu.VMEM((FPW, N), jnp.float32),
        pltpu.VMEM((2, ECH), jnp.int32),
        pltpu.VMEM((2, ECH), jnp.int32),
        pltpu.SemaphoreType.DMA,
        pltpu.SemaphoreType.DMA,
    ),
    compiler_params=_sc_params,
)
def _sc_scatter(pk_hbm, zt_hbm, acc_out,
                z_t, zp_t, acc_t, ebuf0, ebuf1, sem0, sem1):
    c = lax.axis_index("c")
    s = lax.axis_index("s")
    wid = c * NS + s
    ebuf = (ebuf0, ebuf1)
    sem = (sem0, sem1)

    pltpu.sync_copy(zt_hbm.at[wid], z_t)

    zero16 = jnp.zeros((LANES,), jnp.float32)
    half16 = jnp.full((LANES,), 0x8000, jnp.int32)
    mhi16 = jnp.full((LANES,), -0x10000, jnp.int32)

    @pl.loop(0, N // LANES)
    def _zero_and_pack(i):
        sl = pl.ds(i * LANES, LANES)
        for f in range(FPW):
            acc_t[f, sl] = zero16
        for p in range(FPW // 2):
            lo = plsc.bitcast(z_t[2 * p, sl], jnp.int32)
            hi = plsc.bitcast(z_t[2 * p + 1, sl], jnp.int32)
            lo = lax.shift_right_logical(lo + half16, 16)
            hi = lax.bitwise_and(hi + half16, mhi16)
            zp_t[p, sl] = lax.bitwise_or(hi, lo)

    def issue_load(ch, b):
        pltpu.async_copy(pk_hbm.at[ch], ebuf[b], sem[b])

    def wait_load(ch, b):
        pltpu.make_async_copy(pk_hbm.at[ch], ebuf[b], sem[b]).wait()

    mask14 = jnp.full((LANES,), (1 << 14) - 1, jnp.int32)
    fidx = [jnp.full((LANES,), f, jnp.int32) for f in range(FPW)]
    pidx = [jnp.full((LANES,), p, jnp.int32) for p in range(FPW // 2)]

    def process(b):
        @pl.loop(0, EGRP, unroll=25)
        def _grp(g):
            sl = pl.ds(g * LANES, LANES)
            sd = ebuf[b][0, sl]
            w = plsc.bitcast(ebuf[b][1, sl], jnp.float32)
            src = lax.bitwise_and(sd, mask14)
            dst = lax.shift_right_logical(sd, 14)
            for p in range(FPW // 2):
                pv = plsc.load_gather(zp_t, [pidx[p], src])
                zlo = plsc.bitcast(lax.shift_left(pv, 16), jnp.float32)
                zhi = plsc.bitcast(lax.bitwise_and(pv, mhi16), jnp.float32)
                plsc.addupdate_scatter(acc_t, [fidx[2 * p], dst], zlo * w)
                plsc.addupdate_scatter(acc_t, [fidx[2 * p + 1], dst], zhi * w)

    issue_load(0, 0)
    issue_load(1, 1)

    @pl.loop(0, NECH // 2)
    def _pairs(jj):
        c0 = 2 * jj
        wait_load(c0, 0)
        process(0)

        @pl.when(c0 + 2 < NECH)
        def _pf0():
            issue_load(c0 + 2, 0)

        wait_load(c0 + 1, 1)
        process(1)

        @pl.when(c0 + 3 < NECH)
        def _pf1():
            issue_load(c0 + 3, 1)

    pltpu.sync_copy(acc_t, acc_out.at[wid])


def _fin_body(acc_ref, den_ref, out_ref):
    a = jnp.reshape(acc_ref[...], (D, N))
    den = jnp.reshape(den_ref[...], (NW, N))
    ones = jnp.ones((NW, 1), jnp.float32)
    d = lax.dot_general(den, ones, (((0,), (0,)), ((), ())),
                        preferred_element_type=jnp.float32)
    d = jnp.where(d > 0.0, d, 1.0)
    out_ref[...] = a.T / d


def _tc_fin(acc, den):
    return pl.pallas_call(
        _fin_body,
        out_shape=jax.ShapeDtypeStruct((N, D), jnp.float32),
    )(acc, den)


def kernel(h, edge_index, W, a):
    zt, s12 = _tc_prep(h, W, a)
    pk, den = _sc_weights(edge_index, s12)
    acc = _sc_scatter(pk, zt)
    return _tc_fin(acc, den)


# speedup vs baseline: 1.0038x; 1.0038x over previous
import functools

import jax
import jax.numpy as jnp
from jax import lax
from jax.experimental import pallas as pl
from jax.experimental.pallas import tpu as pltpu
from jax.experimental.pallas import tpu_sc as plsc

N = 10000
E = 320000
D = 128

NC = 2
NS = 16
LANES = 16
NW = NC * NS
FPW = D // NW
EPW = E // NW
ECH = 2000
NECH = E // ECH
CPW = EPW // ECH
EGRP = ECH // LANES

_sc_params = pltpu.CompilerParams(
    use_tc_tiling_on_sc=False, needs_layout_passes=False
)

_mesh = plsc.VectorSubcoreMesh(
    core_axis_name="c", subcore_axis_name="s", num_cores=NC, num_subcores=NS
)


def _prep_body(h_ref, w_ref, a_ref, zt_ref, s_ref):
    z = jnp.dot(h_ref[...], w_ref[...], preferred_element_type=jnp.float32)
    zt_ref[...] = jnp.reshape(z.T, (NW, FPW, N))
    a2 = jnp.concatenate([a_ref[0:D, :], a_ref[D:2 * D, :]], axis=1)
    s_ref[...] = jnp.dot(z, a2, preferred_element_type=jnp.float32)


def _tc_prep(h, W, a):
    return pl.pallas_call(
        _prep_body,
        out_shape=[
            jax.ShapeDtypeStruct((NW, FPW, N), jnp.float32),
            jax.ShapeDtypeStruct((N, 2), jnp.float32),
        ],
    )(h, W, a)


@functools.partial(
    pl.kernel,
    out_type=(
        jax.ShapeDtypeStruct((NECH, 2, ECH), jnp.int32),
        jax.ShapeDtypeStruct((NW, 1, N), jnp.float32),
    ),
    mesh=_mesh,
    scratch_types=(
        pltpu.VMEM((ECH,), jnp.int32),
        pltpu.VMEM((ECH,), jnp.int32),
        pltpu.VMEM((2, ECH), jnp.int32),
        pltpu.VMEM((N, 2), jnp.float32),
        pltpu.VMEM((1, N), jnp.float32),
    ),
    compiler_params=_sc_params,
)
def _sc_weights(ei_hbm, s12_hbm, pk_out, den_out,
                src_v, dst_v, pk_v, s12_v, den_v):
    c = lax.axis_index("c")
    s = lax.axis_index("s")
    wid = c * NS + s
    base = wid * EPW

    pltpu.sync_copy(s12_hbm, s12_v)

    zero16 = jnp.zeros((LANES,), jnp.float32)

    @pl.loop(0, N // LANES)
    def _zero_den(i):
        den_v[0, pl.ds(i * LANES, LANES)] = zero16

    col0 = jnp.zeros((LANES,), jnp.int32)
    col1 = jnp.ones((LANES,), jnp.int32)

    for k in range(CPW):
        pltpu.sync_copy(ei_hbm.at[0, pl.ds(base + k * ECH, ECH)], src_v)
        pltpu.sync_copy(ei_hbm.at[1, pl.ds(base + k * ECH, ECH)], dst_v)

        @pl.loop(0, EGRP)
        def _pass1(g):
            sl = pl.ds(g * LANES, LANES)
            si = src_v[sl]
            di = dst_v[sl]
            e = (plsc.load_gather(s12_v, [si, col0])
                 + plsc.load_gather(s12_v, [di, col1]))
            e = jnp.where(e >= 0.0, e, 0.01 * e)
            w = jnp.exp(e)
            pk_v[0, sl] = lax.bitwise_or(si, lax.shift_left(di, 14))
            pk_v[1, sl] = plsc.bitcast(w, jnp.int32)
            plsc.addupdate_scatter(den_v, [col0, di], w)

        pltpu.sync_copy(pk_v, pk_out.at[wid * CPW + k])

    pltpu.sync_copy(den_v, den_out.at[wid])


@functools.partial(
    pl.kernel,
    out_type=jax.ShapeDtypeStruct((NW, FPW, N), jnp.float32),
    mesh=_mesh,
    scratch_types=(
        pltpu.VMEM((FPW, N), jnp.float32),
        pltpu.VMEM((FPW // 2, N), jnp.int32),
        pltpu.VMEM((FPW, N), jnp.float32),
        pltpu.VMEM((2, ECH), jnp.int32),
        pltpu.VMEM((2, ECH), jnp.int32),
        pltpu.SemaphoreType.DMA,
        pltpu.SemaphoreType.DMA,
    ),
    compiler_params=_sc_params,
)
def _sc_scatter(pk_hbm, zt_hbm, acc_out,
                z_t, zp_t, acc_t, ebuf0, ebuf1, sem0, sem1):
    c = lax.axis_index("c")
    s = lax.axis_index("s")
    wid = c * NS + s
    ebuf = (ebuf0, ebuf1)
    sem = (sem0, sem1)

    pltpu.sync_copy(zt_hbm.at[wid], z_t)

    zero16 = jnp.zeros((LANES,), jnp.float32)
    half16 = jnp.full((LANES,), 0x8000, jnp.int32)
    mhi16 = jnp.full((LANES,), -0x10000, jnp.int32)

    @pl.loop(0, N // LANES)
    def _zero_and_pack(i):
        sl = pl.ds(i * LANES, LANES)
        for f in range(FPW):
            acc_t[f, sl] = zero16
        for p in range(FPW // 2):
            lo = plsc.bitcast(z_t[2 * p, sl], jnp.int32)
            hi = plsc.bitcast(z_t[2 * p + 1, sl], jnp.int32)
            lo = lax.shift_right_logical(lo + half16, 16)
            hi = lax.bitwise_and(hi + half16, mhi16)
            zp_t[p, sl] = lax.bitwise_or(hi, lo)

    def issue_load(ch, b):
        pltpu.async_copy(pk_hbm.at[ch], ebuf[b], sem[b])

    def wait_load(ch, b):
        pltpu.make_async_copy(pk_hbm.at[ch], ebuf[b], sem[b]).wait()

    mask14 = jnp.full((LANES,), (1 << 14) - 1, jnp.int32)
    fidx = [jnp.full((LANES,), f, jnp.int32) for f in range(FPW)]
    pidx = [jnp.full((LANES,), p, jnp.int32) for p in range(FPW // 2)]

    def process(b):
        @pl.loop(0, EGRP, unroll=5)
        def _grp(g):
            sl = pl.ds(g * LANES, LANES)
            sd = ebuf[b][0, sl]
            w = plsc.bitcast(ebuf[b][1, sl], jnp.float32)
            src = lax.bitwise_and(sd, mask14)
            dst = lax.shift_right_logical(sd, 14)
            for p in range(FPW // 2):
                pv = plsc.load_gather(zp_t, [pidx[p], src])
                zlo = plsc.bitcast(lax.shift_left(pv, 16), jnp.float32)
                zhi = plsc.bitcast(lax.bitwise_and(pv, mhi16), jnp.float32)
                plsc.addupdate_scatter(acc_t, [fidx[2 * p], dst], zlo * w)
                plsc.addupdate_scatter(acc_t, [fidx[2 * p + 1], dst], zhi * w)

    issue_load(0, 0)
    issue_load(1, 1)

    @pl.loop(0, NECH // 2)
    def _pairs(jj):
        c0 = 2 * jj
        wait_load(c0, 0)
        process(0)

        @pl.when(c0 + 2 < NECH)
        def _pf0():
            issue_load(c0 + 2, 0)

        wait_load(c0 + 1, 1)
        process(1)

        @pl.when(c0 + 3 < NECH)
        def _pf1():
            issue_load(c0 + 3, 1)

    pltpu.sync_copy(acc_t, acc_out.at[wid])


def _fin_body(acc_ref, den_ref, out_ref):
    a = jnp.reshape(acc_ref[...], (D, N))
    den = jnp.reshape(den_ref[...], (NW, N))
    ones = jnp.ones((NW, 1), jnp.float32)
    d = lax.dot_general(den, ones, (((0,), (0,)), ((), ())),
                        preferred_element_type=jnp.float32)
    d = jnp.where(d > 0.0, d, 1.0)
    out_ref[...] = a.T / d


def _tc_fin(acc, den):
    return pl.pallas_call(
        _fin_body,
        out_shape=jax.ShapeDtypeStruct((N, D), jnp.float32),
    )(acc, den)


def kernel(h, edge_index, W, a):
    zt, s12 = _tc_prep(h, W, a)
    pk, den = _sc_weights(edge_index, s12)
    acc = _sc_scatter(pk, zt)
    return _tc_fin(acc, den)
